# baseline (speedup 1.0000x reference)
import functools

import jax
import jax.numpy as jnp
from jax import lax
from jax.experimental import pallas as pl
from jax.experimental.pallas import tpu as pltpu
from jax.experimental.pallas import tpu_sc as plsc

B = 16384
D = 32
NC = 2
NS = 16
NW = NC * NS
BPW = B // NW
L = 16
TILE_WORDS = 1024


def _sc_stage(inp_t, uemb_t, user_bias_flat, bemb_t, book_bias_flat):
    mesh = plsc.VectorSubcoreMesh(core_axis_name="c", subcore_axis_name="s")

    @functools.partial(
        pl.kernel,
        out_type=(
            jax.ShapeDtypeStruct((NW * L,), jnp.float32),
            jax.ShapeDtypeStruct((B,), jnp.float32),
        ),
        mesh=mesh,
        compiler_params=pltpu.CompilerParams(
            needs_layout_passes=False, disable_bounds_checks=True),
        scratch_types=[
            pltpu.VMEM((BPW,), jnp.int32),
            pltpu.VMEM((BPW,), jnp.int32),
            pltpu.VMEM((BPW,), jnp.int32),
            pltpu.VMEM((BPW,), jnp.int32),
            [pltpu.VMEM((BPW,), jnp.float32) for _ in range(D)],
            [pltpu.VMEM((BPW,), jnp.float32) for _ in range(D)],
            pltpu.VMEM((BPW,), jnp.float32),
            pltpu.VMEM((BPW,), jnp.float32),
            pltpu.VMEM((BPW,), jnp.float32),
            pltpu.VMEM((L,), jnp.float32),
            pltpu.SemaphoreType.DMA,
            pltpu.SemaphoreType.DMA,
            pltpu.SemaphoreType.DMA,
            pltpu.SemaphoreType.DMA,
        ],
    )
    def k(inp_hbm, uemb_hbm, ubias_hbm, bemb_hbm, bbias_hbm,
          part_hbm, t_hbm,
          uidx_v, bidx_v, ubase_v, bbase_v, uvals, bvals,
          ub_v, bb_v, t_v, acc_v,
          sem_u, sem_b, sem_ub, sem_bb):
        wid = lax.axis_index("s") * NC + lax.axis_index("c")
        base = wid * BPW
        ci = pltpu.async_copy(inp_hbm.at[0, pl.ds(base, BPW)], uidx_v, sem_ub)
        cj = pltpu.async_copy(inp_hbm.at[1, pl.ds(base, BPW)], bidx_v, sem_bb)
        ci.wait()
        cj.wait()

        lane_mask = jnp.full((L,), 127, jnp.int32)

        def obody(j, carry):
            sl = pl.ds(j * L, L)
            ui = uidx_v[sl]
            bi = bidx_v[sl]
            ubase_v[sl] = lax.shift_left(
                lax.shift_right_logical(ui, 7), 10) + (ui & lane_mask)
            bbase_v[sl] = lax.shift_left(
                lax.shift_right_logical(bi, 7), 10) + (bi & lane_mask)
            return carry

        lax.fori_loop(0, BPW // L, obody, 0)

        cus, cbs = [], []
        for f in range(D):
            cus.append(pltpu.async_copy(
                uemb_hbm.at[f, pl.ds(0, 128)].at[ubase_v],
                uvals[f], sem_u))
            cbs.append(pltpu.async_copy(
                bemb_hbm.at[f, pl.ds(0, 128)].at[bbase_v],
                bvals[f], sem_b))
        cub = pltpu.async_copy(ubias_hbm.at[0].at[uidx_v], ub_v, sem_ub)
        cbb = pltpu.async_copy(bbias_hbm.at[0].at[bidx_v], bb_v, sem_bb)

        cub.wait()
        cbb.wait()

        def tbody(j, carry):
            sl = pl.ds(j * L, L)
            t_v[sl] = ub_v[sl] + bb_v[sl]
            return carry

        lax.fori_loop(0, BPW // L, tbody, 0)
        pltpu.sync_copy(t_v, t_hbm.at[pl.ds(base, BPW)])

        acc = jnp.zeros((L,), jnp.float32)
        for f in range(D):
            cus[f].wait()
            cbs[f].wait()

            def jbody(j, acc, _f=f):
                sl = pl.ds(j * L, L)
                return acc + uvals[_f][sl] * bvals[_f][sl]

            acc = lax.fori_loop(0, BPW // L, jbody, acc)
        acc_v[...] = acc
        pltpu.sync_copy(acc_v, part_hbm.at[pl.ds(wid * L, L)])

    return k(inp_t, uemb_t, user_bias_flat, bemb_t, book_bias_flat)


def _tc_finish(partials, t):
    def body(p_ref, t_ref, o_ref):
        s = jnp.sum(p_ref[...])
        o_ref[...] = jax.nn.sigmoid(t_ref[...] + s)

    out = pl.pallas_call(
        body,
        out_shape=jax.ShapeDtypeStruct((128, 128), jnp.float32),
    )(partials.reshape(4, 128), t.reshape(128, 128))
    return out.reshape(B, 1)


def kernel(inputs, user_embedding, user_bias, book_embedding, book_bias):
    partials, t = _sc_stage(
        inputs.T.astype(jnp.int32),
        user_embedding.T, user_bias.T,
        book_embedding.T, book_bias.T,
    )
    return _tc_finish(partials, t)


# speedup vs baseline: 1.0160x; 1.0160x over previous
import functools

import jax
import jax.numpy as jnp
from jax import lax
from jax.experimental import pallas as pl
from jax.experimental.pallas import tpu as pltpu
from jax.experimental.pallas import tpu_sc as plsc

B = 16384
D = 32
NC = 2
NS = 16
NW = NC * NS
BPW = B // NW
L = 16
TILE_WORDS = 1024


def _sc_stage(inp_t, uemb_t, user_bias_flat, bemb_t, book_bias_flat):
    mesh = plsc.VectorSubcoreMesh(core_axis_name="c", subcore_axis_name="s")

    @functools.partial(
        pl.kernel,
        out_type=(
            jax.ShapeDtypeStruct((NW * L,), jnp.float32),
            jax.ShapeDtypeStruct((B,), jnp.float32),
        ),
        mesh=mesh,
        compiler_params=pltpu.CompilerParams(
            needs_layout_passes=False, disable_bounds_checks=True),
        scratch_types=[
            pltpu.VMEM((BPW,), jnp.int32),
            pltpu.VMEM((BPW,), jnp.int32),
            pltpu.VMEM((BPW,), jnp.int32),
            pltpu.VMEM((BPW,), jnp.int32),
            pltpu.VMEM((8 * BPW,), jnp.int32),
            pltpu.VMEM((8 * BPW,), jnp.int32),
            [pltpu.VMEM((8 * BPW,), jnp.float32) for _ in range(4)],
            [pltpu.VMEM((8 * BPW,), jnp.float32) for _ in range(4)],
            pltpu.VMEM((BPW,), jnp.float32),
            pltpu.VMEM((BPW,), jnp.float32),
            pltpu.VMEM((BPW,), jnp.float32),
            pltpu.VMEM((L,), jnp.float32),
            pltpu.SemaphoreType.DMA,
            pltpu.SemaphoreType.DMA,
            pltpu.SemaphoreType.DMA,
            pltpu.SemaphoreType.DMA,
        ],
    )
    def k(inp_hbm, uemb_hbm, ubias_hbm, bemb_hbm, bbias_hbm,
          part_hbm, t_hbm,
          uidx_v, bidx_v, ubase_v, bbase_v, uexp_v, bexp_v, uvals, bvals,
          ub_v, bb_v, t_v, acc_v,
          sem_u, sem_b, sem_ub, sem_bb):
        wid = lax.axis_index("s") * NC + lax.axis_index("c")
        base = wid * BPW
        ci = pltpu.async_copy(inp_hbm.at[0, pl.ds(base, BPW)], uidx_v, sem_ub)
        cj = pltpu.async_copy(inp_hbm.at[1, pl.ds(base, BPW)], bidx_v, sem_bb)
        ci.wait()
        cj.wait()

        lane_mask = jnp.full((L,), 127, jnp.int32)

        def obody(j, carry):
            sl = pl.ds(j * L, L)
            ui = uidx_v[sl]
            bi = bidx_v[sl]
            ubase_v[sl] = lax.shift_left(
                lax.shift_right_logical(ui, 7), 10) + (ui & lane_mask)
            bbase_v[sl] = lax.shift_left(
                lax.shift_right_logical(bi, 7), 10) + (bi & lane_mask)
            return carry

        lax.fori_loop(0, BPW // L, obody, 0)

        def ebody(j, carry):
            dl = pl.ds(j * L, L)
            sval = lax.shift_left(lax.shift_right_logical(j, 5), 7)
            sl = pl.ds((j & 31) * L, L)
            uexp_v[dl] = ubase_v[sl] + sval
            bexp_v[dl] = bbase_v[sl] + sval
            return carry

        lax.fori_loop(0, 8 * BPW // L, ebody, 0)

        cus, cbs = [], []
        for g in range(4):
            cus.append(pltpu.async_copy(
                uemb_hbm.at[8 * g, pl.ds(0, 128)].at[uexp_v],
                uvals[g], sem_u))
            cbs.append(pltpu.async_copy(
                bemb_hbm.at[8 * g, pl.ds(0, 128)].at[bexp_v],
                bvals[g], sem_b))
        cub = pltpu.async_copy(ubias_hbm.at[0].at[uidx_v], ub_v, sem_ub)
        cbb = pltpu.async_copy(bbias_hbm.at[0].at[bidx_v], bb_v, sem_bb)

        cub.wait()
        cbb.wait()

        def tbody(j, carry):
            sl = pl.ds(j * L, L)
            t_v[sl] = ub_v[sl] + bb_v[sl]
            return carry

        lax.fori_loop(0, BPW // L, tbody, 0)
        pltpu.sync_copy(t_v, t_hbm.at[pl.ds(base, BPW)])

        for c in cus:
            c.wait()
        for c in cbs:
            c.wait()

        def jbody(j, acc):
            for g in range(4):
                for kk in range(8):
                    sl = pl.ds(kk * BPW + j * L, L)
                    acc = acc + uvals[g][sl] * bvals[g][sl]
            return acc

        acc = lax.fori_loop(0, BPW // L, jbody, jnp.zeros((L,), jnp.float32))
        acc_v[...] = acc
        pltpu.sync_copy(acc_v, part_hbm.at[pl.ds(wid * L, L)])

    return k(inp_t, uemb_t, user_bias_flat, bemb_t, book_bias_flat)


def _tc_finish(partials, t):
    def body(p_ref, t_ref, o_ref):
        s = jnp.sum(p_ref[...])
        o_ref[...] = jax.nn.sigmoid(t_ref[...] + s)

    out = pl.pallas_call(
        body,
        out_shape=jax.ShapeDtypeStruct((128, 128), jnp.float32),
    )(partials.reshape(4, 128), t.reshape(128, 128))
    return out.reshape(B, 1)


def kernel(inputs, user_embedding, user_bias, book_embedding, book_bias):
    partials, t = _sc_stage(
        inputs.T.astype(jnp.int32),
        user_embedding.T, user_bias.T,
        book_embedding.T, book_bias.T,
    )
    return _tc_finish(partials, t)


# speedup vs baseline: 1.0452x; 1.0287x over previous
import functools

import jax
import jax.numpy as jnp
from jax import lax
from jax.experimental import pallas as pl
from jax.experimental.pallas import tpu as pltpu
from jax.experimental.pallas import tpu_sc as plsc

B = 16384
D = 32
NC = 2
NS = 16
NW = NC * NS
BPW = B // NW
L = 16
TILE_WORDS = 1024


def _sc_stage(inp_t, uemb_t, user_bias_flat, bemb_t, book_bias_flat):
    mesh = plsc.VectorSubcoreMesh(core_axis_name="c", subcore_axis_name="s")

    @functools.partial(
        pl.kernel,
        out_type=(
            jax.ShapeDtypeStruct((NW * L,), jnp.float32),
            jax.ShapeDtypeStruct((B,), jnp.float32),
        ),
        mesh=mesh,
        compiler_params=pltpu.CompilerParams(
            needs_layout_passes=False, disable_bounds_checks=True),
        scratch_types=[
            pltpu.VMEM((BPW,), jnp.int32),
            pltpu.VMEM((BPW,), jnp.int32),
            pltpu.VMEM((BPW,), jnp.int32),
            pltpu.VMEM((BPW,), jnp.int32),
            [pltpu.VMEM((BPW,), jnp.float32) for _ in range(D)],
            [pltpu.VMEM((BPW,), jnp.float32) for _ in range(D)],
            pltpu.VMEM((BPW,), jnp.float32),
            pltpu.VMEM((BPW,), jnp.float32),
            pltpu.VMEM((BPW,), jnp.float32),
            pltpu.VMEM((L,), jnp.float32),
            pltpu.SemaphoreType.DMA,
            pltpu.SemaphoreType.DMA,
            pltpu.SemaphoreType.DMA,
            pltpu.SemaphoreType.DMA,
        ],
    )
    def k(inp_hbm, uemb_hbm, ubias_hbm, bemb_hbm, bbias_hbm,
          part_hbm, t_hbm,
          uidx_v, bidx_v, ubase_v, bbase_v, uvals, bvals,
          ub_v, bb_v, t_v, acc_v,
          sem_u, sem_b, sem_ub, sem_bb):
        wid = lax.axis_index("s") * NC + lax.axis_index("c")
        base = wid * BPW
        ci = pltpu.async_copy(inp_hbm.at[0, pl.ds(base, BPW)], uidx_v, sem_ub)
        cj = pltpu.async_copy(inp_hbm.at[1, pl.ds(base, BPW)], bidx_v, sem_bb)
        ci.wait()
        cj.wait()

        lane_mask = jnp.full((L,), 127, jnp.int32)

        def obody(j, carry):
            sl = pl.ds(j * L, L)
            ui = uidx_v[sl]
            bi = bidx_v[sl]
            ubase_v[sl] = lax.shift_left(
                lax.shift_right_logical(ui, 7), 10) + (ui & lane_mask)
            bbase_v[sl] = lax.shift_left(
                lax.shift_right_logical(bi, 7), 10) + (bi & lane_mask)
            return carry

        lax.fori_loop(0, BPW // L, obody, 0)

        cus, cbs = [], []
        for f in range(D):
            cus.append(pltpu.async_copy(
                uemb_hbm.at[f, pl.ds(0, 128)].at[ubase_v],
                uvals[f], sem_u))
            cbs.append(pltpu.async_copy(
                bemb_hbm.at[f, pl.ds(0, 128)].at[bbase_v],
                bvals[f], sem_b))
        cub = pltpu.async_copy(ubias_hbm.at[0].at[uidx_v], ub_v, sem_ub)
        cbb = pltpu.async_copy(bbias_hbm.at[0].at[bidx_v], bb_v, sem_bb)

        cub.wait()
        cbb.wait()

        def tbody(j, carry):
            sl = pl.ds(j * L, L)
            t_v[sl] = ub_v[sl] + bb_v[sl]
            return carry

        lax.fori_loop(0, BPW // L, tbody, 0)
        pltpu.sync_copy(t_v, t_hbm.at[pl.ds(base, BPW)])

        for c in cus:
            c.wait()
        for c in cbs:
            c.wait()

        def jbody(j, acc):
            sl = pl.ds(j * L, L)
            for f in range(D):
                acc = acc + uvals[f][sl] * bvals[f][sl]
            return acc

        acc = lax.fori_loop(0, BPW // L, jbody, jnp.zeros((L,), jnp.float32))
        acc_v[...] = acc
        pltpu.sync_copy(acc_v, part_hbm.at[pl.ds(wid * L, L)])

    return k(inp_t, uemb_t, user_bias_flat, bemb_t, book_bias_flat)


def _tc_finish(partials, t):
    def body(p_ref, t_ref, o_ref):
        s = jnp.sum(p_ref[...])
        o_ref[...] = jax.nn.sigmoid(t_ref[...] + s)

    out = pl.pallas_call(
        body,
        out_shape=jax.ShapeDtypeStruct((128, 128), jnp.float32),
    )(partials.reshape(4, 128), t.reshape(128, 128))
    return out.reshape(B, 1)


def kernel(inputs, user_embedding, user_bias, book_embedding, book_bias):
    partials, t = _sc_stage(
        inputs.T.astype(jnp.int32),
        user_embedding.T, user_bias.T,
        book_embedding.T, book_bias.T,
    )
    return _tc_finish(partials, t)
